# baseline (speedup 1.0000x reference)
import jax
import jax.numpy as jnp
from jax import lax
from jax.experimental import pallas as pl
from jax.experimental.pallas import tpu as pltpu
from jax.experimental.pallas import tpu_sc as plsc

N = 131072
N2 = 2 * N
NW = 16
CHUNK = N2 // NW
VREGS = CHUNK // 16
NPASS = 7
RADIX = 32
PAD = 128
MINI = -2147483648


def _srl(v, sh):
    return lax.shift_right_logical(v, sh)


def _body(x_hbm, y_hbm, xw_hbm, yw_hbm, out_hbm,
          key_v, val_v, oidx_v, kstage_v, vstage_v, hist_v, base_v, tmp_v, itmp_v, scal_v,
          dstk_s, dstv_s, grid_s, part_s, dma_sem):
    wid = lax.axis_index("s")
    base = wid * CHUNK
    lane = lax.iota(jnp.int32, 16)

    half = wid < (NW // 2)
    src_off = jnp.where(half, base, base - N)

    @pl.when(half)
    def _():
        pltpu.sync_copy(x_hbm.at[pl.ds(src_off, CHUNK)], vstage_v)
        pltpu.sync_copy(xw_hbm.at[pl.ds(src_off, CHUNK)], val_v)

    @pl.when(jnp.logical_not(half))
    def _():
        pltpu.sync_copy(y_hbm.at[pl.ds(src_off, CHUNK)], vstage_v)
        pltpu.sync_copy(yw_hbm.at[pl.ds(src_off, CHUNK)], val_v)

    sign = jnp.where(half, 1.0, -1.0)

    @pl.loop(0, VREGS, unroll=4)
    def _(i):
        zb = plsc.bitcast(vstage_v[pl.ds(i * 16, 16)], jnp.int32)
        mono = jnp.where(zb < 0, ~zb, zb ^ jnp.int32(MINI))
        key_v[pl.ds(i * 16, 16)] = mono
        val_v[pl.ds(i * 16, 16)] = val_v[pl.ds(i * 16, 16)] * sign

    wsum = lax.fori_loop(
        0, VREGS, lambda i, a: a + val_v[pl.ds(i * 16, 16)],
        jnp.zeros((16,), jnp.float32))
    tmp_v[pl.ds(0, 16)] = jnp.zeros((16,), jnp.float32) + jnp.sum(wsum) * sign
    pltpu.sync_copy(tmp_v.at[pl.ds(0, 16)], part_s.at[pl.ds(wid * 16, 16)])

    pltpu.sync_copy(key_v.at[pl.ds(0, CHUNK)], dstk_s.at[pl.ds(base, CHUNK)])
    pltpu.sync_copy(val_v, dstv_s.at[pl.ds(base, CHUNK)])
    plsc.subcore_barrier()

    def radix_pass(p):
        sh = 5 * p
        pltpu.sync_copy(dstk_s.at[pl.ds(base, CHUNK)], key_v.at[pl.ds(0, CHUNK)])
        pltpu.sync_copy(dstv_s.at[pl.ds(base, CHUNK)], val_v)

        @pl.loop(0, RADIX)
        def _(i):
            hist_v[pl.ds(i * 16, 16)] = jnp.zeros((16,), jnp.int32)

        ones = jnp.ones((16,), jnp.int32)

        @pl.loop(0, VREGS, unroll=4)
        def _(i):
            k = key_v[pl.ds(i * 16, 16)]
            d = _srl(k, sh) & 31
            plsc.addupdate_scatter(hist_v, [lane * 32 + d], ones)

        clo = jnp.zeros((16,), jnp.int32)
        chi = jnp.zeros((16,), jnp.int32)
        for l in range(16):
            clo = clo + hist_v[pl.ds(l * 32, 16)]
            chi = chi + hist_v[pl.ds(l * 32 + 16, 16)]
        itmp_v[pl.ds(0, 16)] = clo
        itmp_v[pl.ds(16, 16)] = chi
        pltpu.sync_copy(itmp_v.at[pl.ds(0, 32)], grid_s.at[pl.ds(wid * 32, 32)])
        plsc.subcore_barrier()

        pltpu.sync_copy(grid_s, itmp_v)
        tot_lo = jnp.zeros((16,), jnp.int32)
        tot_hi = jnp.zeros((16,), jnp.int32)
        bef_lo = jnp.zeros((16,), jnp.int32)
        bef_hi = jnp.zeros((16,), jnp.int32)
        for w in range(NW):
            g_lo = itmp_v[pl.ds(w * 32, 16)]
            g_hi = itmp_v[pl.ds(w * 32 + 16, 16)]
            tot_lo = tot_lo + g_lo
            tot_hi = tot_hi + g_hi
            m = w < wid
            bef_lo = bef_lo + jnp.where(m, g_lo, 0)
            bef_hi = bef_hi + jnp.where(m, g_hi, 0)
        ex_lo = plsc.cumsum(tot_lo) - tot_lo
        ex_hi = plsc.cumsum(tot_hi) - tot_hi + jnp.sum(tot_lo)
        base_v[pl.ds(0, 16)] = ex_lo + bef_lo
        base_v[pl.ds(16, 16)] = ex_hi + bef_hi

        @pl.loop(0, VREGS, unroll=4)
        def _(i):
            kv = key_v[pl.ds(i * 16, 16)]
            vv = val_v[pl.ds(i * 16, 16)]
            d = _srl(kv, sh) & 31
            dk, lv = plsc.sort_key_val(d * 16 + lane, lane)
            ds_ = _srl(dk, 4)
            prev = ds_.at[jnp.maximum(lane - 1, 0)].get(mode="promise_in_bounds")
            nxt = ds_.at[jnp.minimum(lane + 1, 15)].get(mode="promise_in_bounds")
            is_new = (ds_ != prev) | (lane == 0)
            is_last = (ds_ != nxt) | (lane == 15)
            segstart = plsc.cummax(jnp.where(is_new, lane, 0))
            rank = lane - segstart
            packed = (ds_ * 512 + rank * 2) + jnp.where(is_last, 1, 0)
            row = _srl(i, 3)
            col = (i & 7) * 16
            kstage_v[pl.ds(i * 16, 16)] = kv.at[lv].get(mode="promise_in_bounds")
            vstage_v[pl.ds(i * 16, 16)] = vv.at[lv].get(mode="promise_in_bounds")
            oidx_v[row, pl.ds(col, 16)] = packed

        @pl.loop(0, VREGS, unroll=2)
        def _(i):
            row = _srl(i, 3)
            col = (i & 7) * 16
            packed = oidx_v[row, pl.ds(col, 16)]
            ds_ = _srl(packed, 9)
            rank = _srl(packed, 1) & 15
            lastm = (packed & 1) == 1
            pos = plsc.load_gather(base_v, [ds_]) + rank
            plsc.addupdate_scatter(base_v, [ds_], rank + 1, mask=lastm)
            oidx_v[row, pl.ds(col, 16)] = pos

        @pl.loop(0, CHUNK // 128, step=8)
        def _(j):
            handles = []
            for u in range(8):
                handles.append(pltpu.async_copy(
                    kstage_v.at[pl.ds((j + u) * 128, 128)],
                    dstk_s.at[oidx_v.at[j + u]], dma_sem))
                handles.append(pltpu.async_copy(
                    vstage_v.at[pl.ds((j + u) * 128, 128)],
                    dstv_s.at[oidx_v.at[j + u]], dma_sem))
            for h in handles:
                h.wait()

        plsc.subcore_barrier()

    for p in range(NPASS):
        radix_pass(p)

    pltpu.sync_copy(part_s, tmp_v.at[pl.ds(0, 256)])
    sx = jnp.zeros((16,), jnp.float32)
    sy = jnp.zeros((16,), jnp.float32)
    for w in range(NW):
        pv = tmp_v[pl.ds(w * 16, 16)]
        if w < NW // 2:
            sx = sx + pv
        else:
            sy = sy - pv
    rsx = 16.0 / (jnp.zeros((16,), jnp.float32) + jnp.sum(sx))
    rsy = -16.0 / (jnp.zeros((16,), jnp.float32) + jnp.sum(sy))
    plsc.subcore_barrier()

    pltpu.sync_copy(dstk_s.at[pl.ds(base, CHUNK + 16)], key_v)
    pltpu.sync_copy(dstv_s.at[pl.ds(base, CHUNK)], val_v)

    def norm_body(i, a):
        v = val_v[pl.ds(i * 16, 16)]
        v = jnp.where(v >= 0.0, v * rsx, v * rsy)
        val_v[pl.ds(i * 16, 16)] = v
        return a + v

    tloc = lax.fori_loop(0, VREGS, norm_body, jnp.zeros((16,), jnp.float32))
    tmp_v[pl.ds(0, 16)] = jnp.zeros((16,), jnp.float32) + jnp.sum(tloc)
    pltpu.sync_copy(tmp_v.at[pl.ds(0, 16)], part_s.at[pl.ds(wid * 16, 16)])
    plsc.subcore_barrier()

    pltpu.sync_copy(part_s, tmp_v.at[pl.ds(0, 256)])
    off = jnp.zeros((16,), jnp.float32)
    for w in range(NW):
        off = off + jnp.where(w < wid, tmp_v[pl.ds(w * 16, 16)], 0.0)
    off_sc = (jnp.zeros((16,), jnp.float32) + jnp.sum(off)) * 0.0625

    def unkey(vk):
        m = jnp.where(vk < 0, jnp.int32(MINI), jnp.int32(-1))
        return plsc.bitcast(vk ^ m, jnp.float32)

    def red_body(i, carry):
        run, acc = carry
        v = val_v[pl.ds(i * 16, 16)]
        cs = plsc.cumsum(v) + run
        run2 = jnp.zeros((16,), jnp.float32) + cs[15]
        z = unkey(key_v[pl.ds(i * 16, 16)])
        zn = unkey(plsc.load_gather(key_v, [lane + (i * 16 + 1)]))
        gi = (i * 16 + base) + lane
        dz = jnp.where(gi < N2 - 1, zn - z, 0.0)
        return (run2, acc + jnp.abs(cs + off_sc) * dz)

    _, accv = lax.fori_loop(
        0, VREGS, red_body,
        (jnp.zeros((16,), jnp.float32), jnp.zeros((16,), jnp.float32)))
    tmp_v[pl.ds(0, 16)] = jnp.zeros((16,), jnp.float32) + jnp.sum(accv)
    pltpu.sync_copy(tmp_v.at[pl.ds(0, 16)], part_s.at[pl.ds(wid * 16, 16)])
    plsc.subcore_barrier()

    @pl.when(wid == 0)
    def _():
        pltpu.sync_copy(part_s, tmp_v.at[pl.ds(0, 256)])
        t = jnp.zeros((16,), jnp.float32)
        for w in range(NW):
            t = t + tmp_v[pl.ds(w * 16, 16)]
        scal_v[pl.ds(0, 16)] = (jnp.zeros((16,), jnp.float32) + jnp.sum(t)) * 0.0625
        pltpu.sync_copy(scal_v.at[pl.ds(0, 16)], out_hbm)


@jax.jit
def kernel(x, y, x_weights, y_weights):
    mesh = plsc.VectorSubcoreMesh(core_axis_name="c", subcore_axis_name="s",
                                  num_cores=1)
    run = pl.kernel(
        _body,
        out_type=jax.ShapeDtypeStruct((16,), jnp.float32),
        mesh=mesh,
        compiler_params=pltpu.CompilerParams(needs_layout_passes=False),
        scratch_types=[
            pltpu.VMEM((CHUNK + 16,), jnp.int32),
            pltpu.VMEM((CHUNK,), jnp.float32),
            pltpu.VMEM((CHUNK // 128, 128), jnp.int32),
            pltpu.VMEM((CHUNK,), jnp.int32),
            pltpu.VMEM((CHUNK,), jnp.float32),
            pltpu.VMEM((RADIX * 16,), jnp.int32),
            pltpu.VMEM((RADIX,), jnp.int32),
            pltpu.VMEM((512,), jnp.float32),
            pltpu.VMEM((512,), jnp.int32),
            pltpu.VMEM((16,), jnp.float32),
            pltpu.VMEM_SHARED((N2 + PAD,), jnp.int32),
            pltpu.VMEM_SHARED((N2,), jnp.float32),
            pltpu.VMEM_SHARED((NW * 32,), jnp.int32),
            pltpu.VMEM_SHARED((NW * 16,), jnp.float32),
            pltpu.SemaphoreType.DMA,
        ],
    )
    out = run(x, y, x_weights, y_weights)
    return out[0].reshape(())


# speedup vs baseline: 1.3029x; 1.3029x over previous
import jax
import jax.numpy as jnp
from jax import lax
from jax.experimental import pallas as pl
from jax.experimental.pallas import tpu as pltpu
from jax.experimental.pallas import tpu_sc as plsc

N = 131072
N2 = 2 * N
NW = 16
CHUNK = N2 // NW
VREGS = CHUNK // 16
NPASS = 6
RADIX = 64
PAD = 128
MINI = -2147483648


def _srl(v, sh):
    return lax.shift_right_logical(v, sh)


def _body(x_hbm, y_hbm, xw_hbm, yw_hbm, out_hbm,
          key_v, val_v, oidx_v, kstage_v, vstage_v, hist_v, base_v, tmp_v, itmp_v, scal_v,
          dstk_s, dstv_s, grid_s, part_s, dma_sem):
    wid = lax.axis_index("s")
    base = wid * CHUNK
    lane = lax.iota(jnp.int32, 16)

    half = wid < (NW // 2)
    src_off = jnp.where(half, base, base - N)

    @pl.when(half)
    def _():
        pltpu.sync_copy(x_hbm.at[pl.ds(src_off, CHUNK)], vstage_v)
        pltpu.sync_copy(xw_hbm.at[pl.ds(src_off, CHUNK)], val_v)

    @pl.when(jnp.logical_not(half))
    def _():
        pltpu.sync_copy(y_hbm.at[pl.ds(src_off, CHUNK)], vstage_v)
        pltpu.sync_copy(yw_hbm.at[pl.ds(src_off, CHUNK)], val_v)

    sign = jnp.where(half, 1.0, -1.0)

    @pl.loop(0, VREGS, unroll=4)
    def _(i):
        zb = plsc.bitcast(vstage_v[pl.ds(i * 16, 16)], jnp.int32)
        mono = jnp.where(zb < 0, ~zb, zb ^ jnp.int32(MINI))
        key_v[pl.ds(i * 16, 16)] = mono
        val_v[pl.ds(i * 16, 16)] = val_v[pl.ds(i * 16, 16)] * sign

    wsum = lax.fori_loop(
        0, VREGS, lambda i, a: a + val_v[pl.ds(i * 16, 16)],
        jnp.zeros((16,), jnp.float32))
    tmp_v[pl.ds(0, 16)] = jnp.zeros((16,), jnp.float32) + jnp.sum(wsum) * sign
    pltpu.sync_copy(tmp_v.at[pl.ds(0, 16)], part_s.at[pl.ds(wid * 16, 16)])

    pltpu.sync_copy(key_v.at[pl.ds(0, CHUNK)], dstk_s.at[pl.ds(base, CHUNK)])
    pltpu.sync_copy(val_v, dstv_s.at[pl.ds(base, CHUNK)])
    plsc.subcore_barrier()

    def radix_pass(p):
        sh = 6 * p
        pltpu.sync_copy(dstk_s.at[pl.ds(base, CHUNK)], key_v.at[pl.ds(0, CHUNK)])
        pltpu.sync_copy(dstv_s.at[pl.ds(base, CHUNK)], val_v)

        @pl.loop(0, RADIX)
        def _(i):
            hist_v[pl.ds(i * 16, 16)] = jnp.zeros((16,), jnp.int32)

        ones = jnp.ones((16,), jnp.int32)

        @pl.loop(0, VREGS, unroll=4)
        def _(i):
            k = key_v[pl.ds(i * 16, 16)]
            d = _srl(k, sh) & (RADIX - 1)
            plsc.addupdate_scatter(hist_v, [lane * RADIX + d], ones)

        NG = RADIX // 16
        for g in range(NG):
            cg = jnp.zeros((16,), jnp.int32)
            for l in range(16):
                cg = cg + hist_v[pl.ds(l * RADIX + g * 16, 16)]
            itmp_v[pl.ds(g * 16, 16)] = cg
        pltpu.sync_copy(itmp_v.at[pl.ds(0, RADIX)],
                        grid_s.at[pl.ds(wid * RADIX, RADIX)])
        plsc.subcore_barrier()

        pltpu.sync_copy(grid_s, itmp_v)
        tot = [jnp.zeros((16,), jnp.int32) for _ in range(NG)]
        bef = [jnp.zeros((16,), jnp.int32) for _ in range(NG)]
        for w in range(NW):
            m = w < wid
            for g in range(NG):
                gv = itmp_v[pl.ds(w * RADIX + g * 16, 16)]
                tot[g] = tot[g] + gv
                bef[g] = bef[g] + jnp.where(m, gv, 0)
        run = None
        for g in range(NG):
            ex = plsc.cumsum(tot[g]) - tot[g]
            if run is not None:
                ex = ex + run
            base_v[pl.ds(g * 16, 16)] = ex + bef[g]
            run = jnp.sum(tot[g]) if run is None else run + jnp.sum(tot[g])

        @pl.loop(0, VREGS, unroll=4)
        def _(i):
            kv = key_v[pl.ds(i * 16, 16)]
            vv = val_v[pl.ds(i * 16, 16)]
            d = _srl(kv, sh) & (RADIX - 1)
            dk, lv = plsc.sort_key_val(d * 16 + lane, lane)
            ds_ = _srl(dk, 4)
            prev = ds_.at[jnp.maximum(lane - 1, 0)].get(mode="promise_in_bounds")
            nxt = ds_.at[jnp.minimum(lane + 1, 15)].get(mode="promise_in_bounds")
            is_new = (ds_ != prev) | (lane == 0)
            is_last = (ds_ != nxt) | (lane == 15)
            segstart = plsc.cummax(jnp.where(is_new, lane, 0))
            rank = lane - segstart
            pos = plsc.load_gather(base_v, [ds_]) + rank
            plsc.addupdate_scatter(base_v, [ds_], rank + 1, mask=is_last)
            row = _srl(i, 3)
            col = (i & 7) * 16
            kstage_v[pl.ds(i * 16, 16)] = kv.at[lv].get(mode="promise_in_bounds")
            vstage_v[pl.ds(i * 16, 16)] = vv.at[lv].get(mode="promise_in_bounds")
            oidx_v[row, pl.ds(col, 16)] = pos

        @pl.loop(0, CHUNK // 128, step=8)
        def _(j):
            handles = []
            for u in range(8):
                handles.append(pltpu.async_copy(
                    kstage_v.at[pl.ds((j + u) * 128, 128)],
                    dstk_s.at[oidx_v.at[j + u]], dma_sem))
                handles.append(pltpu.async_copy(
                    vstage_v.at[pl.ds((j + u) * 128, 128)],
                    dstv_s.at[oidx_v.at[j + u]], dma_sem))
            for h in handles:
                h.wait()

        plsc.subcore_barrier()

    for p in range(NPASS):
        radix_pass(p)

    pltpu.sync_copy(part_s, tmp_v.at[pl.ds(0, 256)])
    sx = jnp.zeros((16,), jnp.float32)
    sy = jnp.zeros((16,), jnp.float32)
    for w in range(NW):
        pv = tmp_v[pl.ds(w * 16, 16)]
        if w < NW // 2:
            sx = sx + pv
        else:
            sy = sy - pv
    rsx = 16.0 / (jnp.zeros((16,), jnp.float32) + jnp.sum(sx))
    rsy = -16.0 / (jnp.zeros((16,), jnp.float32) + jnp.sum(sy))
    plsc.subcore_barrier()

    pltpu.sync_copy(dstk_s.at[pl.ds(base, CHUNK + 16)], key_v)
    pltpu.sync_copy(dstv_s.at[pl.ds(base, CHUNK)], val_v)

    def norm_body(i, a):
        v = val_v[pl.ds(i * 16, 16)]
        v = jnp.where(v >= 0.0, v * rsx, v * rsy)
        val_v[pl.ds(i * 16, 16)] = v
        return a + v

    tloc = lax.fori_loop(0, VREGS, norm_body, jnp.zeros((16,), jnp.float32))
    tmp_v[pl.ds(0, 16)] = jnp.zeros((16,), jnp.float32) + jnp.sum(tloc)
    pltpu.sync_copy(tmp_v.at[pl.ds(0, 16)], part_s.at[pl.ds(wid * 16, 16)])
    plsc.subcore_barrier()

    pltpu.sync_copy(part_s, tmp_v.at[pl.ds(0, 256)])
    off = jnp.zeros((16,), jnp.float32)
    for w in range(NW):
        off = off + jnp.where(w < wid, tmp_v[pl.ds(w * 16, 16)], 0.0)
    off_sc = (jnp.zeros((16,), jnp.float32) + jnp.sum(off)) * 0.0625

    def unkey(vk):
        m = jnp.where(vk < 0, jnp.int32(MINI), jnp.int32(-1))
        return plsc.bitcast(vk ^ m, jnp.float32)

    def red_body(i, carry):
        run, acc = carry
        v = val_v[pl.ds(i * 16, 16)]
        cs = plsc.cumsum(v) + run
        run2 = jnp.zeros((16,), jnp.float32) + cs[15]
        z = unkey(key_v[pl.ds(i * 16, 16)])
        zn = unkey(plsc.load_gather(key_v, [lane + (i * 16 + 1)]))
        gi = (i * 16 + base) + lane
        dz = jnp.where(gi < N2 - 1, zn - z, 0.0)
        return (run2, acc + jnp.abs(cs + off_sc) * dz)

    _, accv = lax.fori_loop(
        0, VREGS, red_body,
        (jnp.zeros((16,), jnp.float32), jnp.zeros((16,), jnp.float32)))
    tmp_v[pl.ds(0, 16)] = jnp.zeros((16,), jnp.float32) + jnp.sum(accv)
    pltpu.sync_copy(tmp_v.at[pl.ds(0, 16)], part_s.at[pl.ds(wid * 16, 16)])
    plsc.subcore_barrier()

    @pl.when(wid == 0)
    def _():
        pltpu.sync_copy(part_s, tmp_v.at[pl.ds(0, 256)])
        t = jnp.zeros((16,), jnp.float32)
        for w in range(NW):
            t = t + tmp_v[pl.ds(w * 16, 16)]
        scal_v[pl.ds(0, 16)] = (jnp.zeros((16,), jnp.float32) + jnp.sum(t)) * 0.0625
        pltpu.sync_copy(scal_v.at[pl.ds(0, 16)], out_hbm)


@jax.jit
def kernel(x, y, x_weights, y_weights):
    mesh = plsc.VectorSubcoreMesh(core_axis_name="c", subcore_axis_name="s",
                                  num_cores=1)
    run = pl.kernel(
        _body,
        out_type=jax.ShapeDtypeStruct((16,), jnp.float32),
        mesh=mesh,
        compiler_params=pltpu.CompilerParams(needs_layout_passes=False),
        scratch_types=[
            pltpu.VMEM((CHUNK + 16,), jnp.int32),
            pltpu.VMEM((CHUNK,), jnp.float32),
            pltpu.VMEM((CHUNK // 128, 128), jnp.int32),
            pltpu.VMEM((CHUNK,), jnp.int32),
            pltpu.VMEM((CHUNK,), jnp.float32),
            pltpu.VMEM((RADIX * 16,), jnp.int32),
            pltpu.VMEM((RADIX,), jnp.int32),
            pltpu.VMEM((512,), jnp.float32),
            pltpu.VMEM((NW * RADIX,), jnp.int32),
            pltpu.VMEM((16,), jnp.float32),
            pltpu.VMEM_SHARED((N2 + PAD,), jnp.int32),
            pltpu.VMEM_SHARED((N2,), jnp.float32),
            pltpu.VMEM_SHARED((NW * RADIX,), jnp.int32),
            pltpu.VMEM_SHARED((NW * 16,), jnp.float32),
            pltpu.SemaphoreType.DMA,
        ],
    )
    out = run(x, y, x_weights, y_weights)
    return out[0].reshape(())


# speedup vs baseline: 1.4791x; 1.1353x over previous
import jax
import jax.numpy as jnp
from jax import lax
from jax.experimental import pallas as pl
from jax.experimental.pallas import tpu as pltpu
from jax.experimental.pallas import tpu_sc as plsc

N = 131072
N2 = 2 * N
NW = 16
CHUNK = N2 // NW
VREGS = CHUNK // 16
NPASS = 5
RADIX = 128
PAD = 128
MINI = -2147483648


def _srl(v, sh):
    return lax.shift_right_logical(v, sh)


def _body(x_hbm, y_hbm, xw_hbm, yw_hbm, out_hbm,
          key_v, val_v, oidx_v, kstage_v, vstage_v, hist_v, base_v, tmp_v, itmp_v, scal_v,
          dstk_s, dstv_s, grid_s, part_s, dma_sem):
    wid = lax.axis_index("s")
    base = wid * CHUNK
    lane = lax.iota(jnp.int32, 16)

    half = wid < (NW // 2)
    src_off = jnp.where(half, base, base - N)

    @pl.when(half)
    def _():
        pltpu.sync_copy(x_hbm.at[pl.ds(src_off, CHUNK)], vstage_v)
        pltpu.sync_copy(xw_hbm.at[pl.ds(src_off, CHUNK)], val_v)

    @pl.when(jnp.logical_not(half))
    def _():
        pltpu.sync_copy(y_hbm.at[pl.ds(src_off, CHUNK)], vstage_v)
        pltpu.sync_copy(yw_hbm.at[pl.ds(src_off, CHUNK)], val_v)

    sign = jnp.where(half, 1.0, -1.0)

    @pl.loop(0, VREGS, unroll=4)
    def _(i):
        zb = plsc.bitcast(vstage_v[pl.ds(i * 16, 16)], jnp.int32)
        mono = jnp.where(zb < 0, ~zb, zb ^ jnp.int32(MINI))
        key_v[pl.ds(i * 16, 16)] = mono
        val_v[pl.ds(i * 16, 16)] = val_v[pl.ds(i * 16, 16)] * sign

    wsum = lax.fori_loop(
        0, VREGS, lambda i, a: a + val_v[pl.ds(i * 16, 16)],
        jnp.zeros((16,), jnp.float32))
    tmp_v[pl.ds(0, 16)] = jnp.zeros((16,), jnp.float32) + jnp.sum(wsum) * sign
    pltpu.sync_copy(tmp_v.at[pl.ds(0, 16)], part_s.at[pl.ds(wid * 16, 16)])

    pltpu.sync_copy(key_v.at[pl.ds(0, CHUNK)], dstk_s.at[pl.ds(base, CHUNK)])
    pltpu.sync_copy(val_v, dstv_s.at[pl.ds(base, CHUNK)])
    plsc.subcore_barrier()

    def radix_pass(p):
        sh = 7 * p
        pltpu.sync_copy(dstk_s.at[pl.ds(base, CHUNK)], key_v.at[pl.ds(0, CHUNK)])
        pltpu.sync_copy(dstv_s.at[pl.ds(base, CHUNK)], val_v)

        @pl.loop(0, RADIX)
        def _(i):
            hist_v[pl.ds(i * 16, 16)] = jnp.zeros((16,), jnp.int32)

        ones = jnp.ones((16,), jnp.int32)

        @pl.loop(0, VREGS, unroll=4)
        def _(i):
            k = key_v[pl.ds(i * 16, 16)]
            d = _srl(k, sh) & (RADIX - 1)
            plsc.addupdate_scatter(hist_v, [lane * RADIX + d], ones)

        NG = RADIX // 16
        for g in range(NG):
            cg = jnp.zeros((16,), jnp.int32)
            for l in range(16):
                cg = cg + hist_v[pl.ds(l * RADIX + g * 16, 16)]
            itmp_v[pl.ds(g * 16, 16)] = cg
        pltpu.sync_copy(itmp_v.at[pl.ds(0, RADIX)],
                        grid_s.at[pl.ds(wid * RADIX, RADIX)])
        plsc.subcore_barrier()

        pltpu.sync_copy(grid_s, itmp_v)
        tot = [jnp.zeros((16,), jnp.int32) for _ in range(NG)]
        bef = [jnp.zeros((16,), jnp.int32) for _ in range(NG)]
        for w in range(NW):
            m = w < wid
            for g in range(NG):
                gv = itmp_v[pl.ds(w * RADIX + g * 16, 16)]
                tot[g] = tot[g] + gv
                bef[g] = bef[g] + jnp.where(m, gv, 0)
        run = None
        for g in range(NG):
            ex = plsc.cumsum(tot[g]) - tot[g]
            if run is not None:
                ex = ex + run
            base_v[pl.ds(g * 16, 16)] = ex + bef[g]
            run = jnp.sum(tot[g]) if run is None else run + jnp.sum(tot[g])

        @pl.loop(0, VREGS, unroll=4)
        def _(i):
            kv = key_v[pl.ds(i * 16, 16)]
            vv = val_v[pl.ds(i * 16, 16)]
            d = _srl(kv, sh) & (RADIX - 1)
            dk, lv = plsc.sort_key_val(d * 16 + lane, lane)
            ds_ = _srl(dk, 4)
            prev = ds_.at[jnp.maximum(lane - 1, 0)].get(mode="promise_in_bounds")
            nxt = ds_.at[jnp.minimum(lane + 1, 15)].get(mode="promise_in_bounds")
            is_new = (ds_ != prev) | (lane == 0)
            is_last = (ds_ != nxt) | (lane == 15)
            segstart = plsc.cummax(jnp.where(is_new, lane, 0))
            rank = lane - segstart
            pos = plsc.load_gather(base_v, [ds_]) + rank
            plsc.addupdate_scatter(base_v, [ds_], rank + 1, mask=is_last)
            row = _srl(i, 3)
            col = (i & 7) * 16
            kstage_v[pl.ds(i * 16, 16)] = kv.at[lv].get(mode="promise_in_bounds")
            vstage_v[pl.ds(i * 16, 16)] = vv.at[lv].get(mode="promise_in_bounds")
            oidx_v[row, pl.ds(col, 16)] = pos

        @pl.loop(0, CHUNK // 128, step=8)
        def _(j):
            handles = []
            for u in range(8):
                handles.append(pltpu.async_copy(
                    kstage_v.at[pl.ds((j + u) * 128, 128)],
                    dstk_s.at[oidx_v.at[j + u]], dma_sem))
                handles.append(pltpu.async_copy(
                    vstage_v.at[pl.ds((j + u) * 128, 128)],
                    dstv_s.at[oidx_v.at[j + u]], dma_sem))
            for h in handles:
                h.wait()

        plsc.subcore_barrier()

    for p in range(NPASS):
        radix_pass(p)

    pltpu.sync_copy(part_s, tmp_v.at[pl.ds(0, 256)])
    sx = jnp.zeros((16,), jnp.float32)
    sy = jnp.zeros((16,), jnp.float32)
    for w in range(NW):
        pv = tmp_v[pl.ds(w * 16, 16)]
        if w < NW // 2:
            sx = sx + pv
        else:
            sy = sy - pv
    rsx = 16.0 / (jnp.zeros((16,), jnp.float32) + jnp.sum(sx))
    rsy = -16.0 / (jnp.zeros((16,), jnp.float32) + jnp.sum(sy))
    plsc.subcore_barrier()

    pltpu.sync_copy(dstk_s.at[pl.ds(base, CHUNK + 16)], key_v)
    pltpu.sync_copy(dstv_s.at[pl.ds(base, CHUNK)], val_v)

    def norm_body(i, a):
        v = val_v[pl.ds(i * 16, 16)]
        v = jnp.where(v >= 0.0, v * rsx, v * rsy)
        val_v[pl.ds(i * 16, 16)] = v
        return a + v

    tloc = lax.fori_loop(0, VREGS, norm_body, jnp.zeros((16,), jnp.float32))
    tmp_v[pl.ds(0, 16)] = jnp.zeros((16,), jnp.float32) + jnp.sum(tloc)
    pltpu.sync_copy(tmp_v.at[pl.ds(0, 16)], part_s.at[pl.ds(wid * 16, 16)])
    plsc.subcore_barrier()

    pltpu.sync_copy(part_s, tmp_v.at[pl.ds(0, 256)])
    off = jnp.zeros((16,), jnp.float32)
    for w in range(NW):
        off = off + jnp.where(w < wid, tmp_v[pl.ds(w * 16, 16)], 0.0)
    off_sc = (jnp.zeros((16,), jnp.float32) + jnp.sum(off)) * 0.0625

    def unkey(vk):
        m = jnp.where(vk < 0, jnp.int32(MINI), jnp.int32(-1))
        return plsc.bitcast(vk ^ m, jnp.float32)

    def red_body(i, carry):
        run, acc = carry
        v = val_v[pl.ds(i * 16, 16)]
        cs = plsc.cumsum(v) + run
        run2 = jnp.zeros((16,), jnp.float32) + cs[15]
        z = unkey(key_v[pl.ds(i * 16, 16)])
        zn = unkey(plsc.load_gather(key_v, [lane + (i * 16 + 1)]))
        gi = (i * 16 + base) + lane
        dz = jnp.where(gi < N2 - 1, zn - z, 0.0)
        return (run2, acc + jnp.abs(cs + off_sc) * dz)

    _, accv = lax.fori_loop(
        0, VREGS, red_body,
        (jnp.zeros((16,), jnp.float32), jnp.zeros((16,), jnp.float32)))
    tmp_v[pl.ds(0, 16)] = jnp.zeros((16,), jnp.float32) + jnp.sum(accv)
    pltpu.sync_copy(tmp_v.at[pl.ds(0, 16)], part_s.at[pl.ds(wid * 16, 16)])
    plsc.subcore_barrier()

    @pl.when(wid == 0)
    def _():
        pltpu.sync_copy(part_s, tmp_v.at[pl.ds(0, 256)])
        t = jnp.zeros((16,), jnp.float32)
        for w in range(NW):
            t = t + tmp_v[pl.ds(w * 16, 16)]
        scal_v[pl.ds(0, 16)] = (jnp.zeros((16,), jnp.float32) + jnp.sum(t)) * 0.0625
        pltpu.sync_copy(scal_v.at[pl.ds(0, 16)], out_hbm)


@jax.jit
def kernel(x, y, x_weights, y_weights):
    mesh = plsc.VectorSubcoreMesh(core_axis_name="c", subcore_axis_name="s",
                                  num_cores=1)
    run = pl.kernel(
        _body,
        out_type=jax.ShapeDtypeStruct((16,), jnp.float32),
        mesh=mesh,
        compiler_params=pltpu.CompilerParams(needs_layout_passes=False),
        scratch_types=[
            pltpu.VMEM((CHUNK + 16,), jnp.int32),
            pltpu.VMEM((CHUNK,), jnp.float32),
            pltpu.VMEM((CHUNK // 128, 128), jnp.int32),
            pltpu.VMEM((CHUNK,), jnp.int32),
            pltpu.VMEM((CHUNK,), jnp.float32),
            pltpu.VMEM((RADIX * 16,), jnp.int32),
            pltpu.VMEM((RADIX,), jnp.int32),
            pltpu.VMEM((512,), jnp.float32),
            pltpu.VMEM((NW * RADIX,), jnp.int32),
            pltpu.VMEM((16,), jnp.float32),
            pltpu.VMEM_SHARED((N2 + PAD,), jnp.int32),
            pltpu.VMEM_SHARED((N2,), jnp.float32),
            pltpu.VMEM_SHARED((NW * RADIX,), jnp.int32),
            pltpu.VMEM_SHARED((NW * 16,), jnp.float32),
            pltpu.SemaphoreType.DMA,
        ],
    )
    out = run(x, y, x_weights, y_weights)
    return out[0].reshape(())


# speedup vs baseline: 1.5079x; 1.0195x over previous
import jax
import jax.numpy as jnp
from jax import lax
from jax.experimental import pallas as pl
from jax.experimental.pallas import tpu as pltpu
from jax.experimental.pallas import tpu_sc as plsc

N = 131072
N2 = 2 * N
NW = 16
CHUNK = N2 // NW
VREGS = CHUNK // 16
NPASS = 5
RADIX = 128
PAD = 128
MINI = -2147483648


def _srl(v, sh):
    return lax.shift_right_logical(v, sh)


def _body(x_hbm, y_hbm, xw_hbm, yw_hbm, out_hbm,
          key_v, val_v, oidx_v, kstage_v, vstage_v, hist_v, base_v, tmp_v, itmp_v, scal_v,
          dstk_s, dstv_s, grid_s, part_s, dma_sem):
    wid = lax.axis_index("s")
    base = wid * CHUNK
    lane = lax.iota(jnp.int32, 16)

    half = wid < (NW // 2)
    src_off = jnp.where(half, base, base - N)

    @pl.when(half)
    def _():
        pltpu.sync_copy(x_hbm.at[pl.ds(src_off, CHUNK)], vstage_v)
        pltpu.sync_copy(xw_hbm.at[pl.ds(src_off, CHUNK)], val_v)

    @pl.when(jnp.logical_not(half))
    def _():
        pltpu.sync_copy(y_hbm.at[pl.ds(src_off, CHUNK)], vstage_v)
        pltpu.sync_copy(yw_hbm.at[pl.ds(src_off, CHUNK)], val_v)

    sign = jnp.where(half, 1.0, -1.0)

    @pl.loop(0, VREGS, unroll=4)
    def _(i):
        zb = plsc.bitcast(vstage_v[pl.ds(i * 16, 16)], jnp.int32)
        mono = jnp.where(zb < 0, ~zb, zb ^ jnp.int32(MINI))
        key_v[pl.ds(i * 16, 16)] = mono
        val_v[pl.ds(i * 16, 16)] = val_v[pl.ds(i * 16, 16)] * sign

    wsum = lax.fori_loop(
        0, VREGS, lambda i, a: a + val_v[pl.ds(i * 16, 16)],
        jnp.zeros((16,), jnp.float32))
    tmp_v[pl.ds(0, 16)] = jnp.zeros((16,), jnp.float32) + jnp.sum(wsum) * sign
    pltpu.sync_copy(tmp_v.at[pl.ds(0, 16)], part_s.at[pl.ds(wid * 16, 16)])

    pltpu.sync_copy(key_v.at[pl.ds(0, CHUNK)], dstk_s.at[pl.ds(base, CHUNK)])
    pltpu.sync_copy(val_v, dstv_s.at[pl.ds(base, CHUNK)])
    plsc.subcore_barrier()

    def radix_pass(p):
        sh = 7 * p
        h1 = pltpu.async_copy(dstk_s.at[pl.ds(base, CHUNK)],
                              key_v.at[pl.ds(0, CHUNK)], dma_sem)
        h2 = pltpu.async_copy(dstv_s.at[pl.ds(base, CHUNK)], val_v, dma_sem)
        h1.wait()
        h2.wait()

        @pl.loop(0, RADIX)
        def _(i):
            hist_v[pl.ds(i * 16, 16)] = jnp.zeros((16,), jnp.int32)

        ones = jnp.ones((16,), jnp.int32)

        @pl.loop(0, VREGS, unroll=4)
        def _(i):
            k = key_v[pl.ds(i * 16, 16)]
            d = _srl(k, sh) & (RADIX - 1)
            plsc.addupdate_scatter(hist_v, [lane * RADIX + d], ones)

        NG = RADIX // 16
        for g in range(NG):
            cg = jnp.zeros((16,), jnp.int32)
            for l in range(16):
                cg = cg + hist_v[pl.ds(l * RADIX + g * 16, 16)]
            itmp_v[pl.ds(g * 16, 16)] = cg
        pltpu.sync_copy(itmp_v.at[pl.ds(0, RADIX)],
                        grid_s.at[pl.ds(wid * RADIX, RADIX)])
        plsc.subcore_barrier()

        pltpu.sync_copy(grid_s, itmp_v)
        tot = [jnp.zeros((16,), jnp.int32) for _ in range(NG)]
        bef = [jnp.zeros((16,), jnp.int32) for _ in range(NG)]
        for w in range(NW):
            m = w < wid
            for g in range(NG):
                gv = itmp_v[pl.ds(w * RADIX + g * 16, 16)]
                tot[g] = tot[g] + gv
                bef[g] = bef[g] + jnp.where(m, gv, 0)
        run = None
        for g in range(NG):
            ex = plsc.cumsum(tot[g]) - tot[g]
            if run is not None:
                ex = ex + run
            base_v[pl.ds(g * 16, 16)] = ex + bef[g]
            run = jnp.sum(tot[g]) if run is None else run + jnp.sum(tot[g])

        @pl.loop(0, VREGS, unroll=4)
        def _(i):
            kv = key_v[pl.ds(i * 16, 16)]
            vv = val_v[pl.ds(i * 16, 16)]
            d = _srl(kv, sh) & (RADIX - 1)
            dk, lv = plsc.sort_key_val(d * 16 + lane, lane)
            ds_ = _srl(dk, 4)
            prev = ds_.at[jnp.maximum(lane - 1, 0)].get(mode="promise_in_bounds")
            nxt = ds_.at[jnp.minimum(lane + 1, 15)].get(mode="promise_in_bounds")
            is_new = (ds_ != prev) | (lane == 0)
            is_last = (ds_ != nxt) | (lane == 15)
            segstart = plsc.cummax(jnp.where(is_new, lane, 0))
            rank = lane - segstart
            pos = plsc.load_gather(base_v, [ds_]) + rank
            plsc.addupdate_scatter(base_v, [ds_], rank + 1, mask=is_last)
            row = _srl(i, 3)
            col = (i & 7) * 16
            kstage_v[pl.ds(i * 16, 16)] = kv.at[lv].get(mode="promise_in_bounds")
            vstage_v[pl.ds(i * 16, 16)] = vv.at[lv].get(mode="promise_in_bounds")
            oidx_v[row, pl.ds(col, 16)] = pos

        @pl.loop(0, CHUNK // 128, step=16)
        def _(j):
            handles = []
            for u in range(16):
                handles.append(pltpu.async_copy(
                    kstage_v.at[pl.ds((j + u) * 128, 128)],
                    dstk_s.at[oidx_v.at[j + u]], dma_sem))
                handles.append(pltpu.async_copy(
                    vstage_v.at[pl.ds((j + u) * 128, 128)],
                    dstv_s.at[oidx_v.at[j + u]], dma_sem))
            for h in handles:
                h.wait()

        plsc.subcore_barrier()

    for p in range(NPASS):
        radix_pass(p)

    pltpu.sync_copy(part_s, tmp_v.at[pl.ds(0, 256)])
    sx = jnp.zeros((16,), jnp.float32)
    sy = jnp.zeros((16,), jnp.float32)
    for w in range(NW):
        pv = tmp_v[pl.ds(w * 16, 16)]
        if w < NW // 2:
            sx = sx + pv
        else:
            sy = sy - pv
    rsx = 16.0 / (jnp.zeros((16,), jnp.float32) + jnp.sum(sx))
    rsy = -16.0 / (jnp.zeros((16,), jnp.float32) + jnp.sum(sy))
    plsc.subcore_barrier()

    h1 = pltpu.async_copy(dstk_s.at[pl.ds(base, CHUNK + 16)], key_v, dma_sem)
    h2 = pltpu.async_copy(dstv_s.at[pl.ds(base, CHUNK)], val_v, dma_sem)
    h1.wait()
    h2.wait()

    @pl.loop(0, VREGS, init_carry=jnp.zeros((16,), jnp.float32), unroll=4)
    def tloc(i, a):
        v = val_v[pl.ds(i * 16, 16)]
        return a + jnp.where(v >= 0.0, v * rsx, v * rsy)
    tmp_v[pl.ds(0, 16)] = jnp.zeros((16,), jnp.float32) + jnp.sum(tloc)
    pltpu.sync_copy(tmp_v.at[pl.ds(0, 16)], part_s.at[pl.ds(wid * 16, 16)])
    plsc.subcore_barrier()

    pltpu.sync_copy(part_s, tmp_v.at[pl.ds(0, 256)])
    off = jnp.zeros((16,), jnp.float32)
    for w in range(NW):
        off = off + jnp.where(w < wid, tmp_v[pl.ds(w * 16, 16)], 0.0)
    off_sc = (jnp.zeros((16,), jnp.float32) + jnp.sum(off)) * 0.0625

    def unkey(vk):
        m = jnp.where(vk < 0, jnp.int32(MINI), jnp.int32(-1))
        return plsc.bitcast(vk ^ m, jnp.float32)

    @pl.loop(0, VREGS,
             init_carry=(jnp.zeros((16,), jnp.float32),
                         jnp.zeros((16,), jnp.float32)),
             unroll=2)
    def _red(i, carry):
        run, acc = carry
        v = val_v[pl.ds(i * 16, 16)]
        vn = jnp.where(v >= 0.0, v * rsx, v * rsy)
        cs = plsc.cumsum(vn) + run
        run2 = jnp.zeros((16,), jnp.float32) + cs[15]
        z = unkey(key_v[pl.ds(i * 16, 16)])
        zn = unkey(plsc.load_gather(key_v, [lane + (i * 16 + 1)]))
        gi = (i * 16 + base) + lane
        dz = jnp.where(gi < N2 - 1, zn - z, 0.0)
        return (run2, acc + jnp.abs(cs + off_sc) * dz)

    _, accv = _red
    tmp_v[pl.ds(0, 16)] = jnp.zeros((16,), jnp.float32) + jnp.sum(accv)
    pltpu.sync_copy(tmp_v.at[pl.ds(0, 16)], part_s.at[pl.ds(wid * 16, 16)])
    plsc.subcore_barrier()

    @pl.when(wid == 0)
    def _():
        pltpu.sync_copy(part_s, tmp_v.at[pl.ds(0, 256)])
        t = jnp.zeros((16,), jnp.float32)
        for w in range(NW):
            t = t + tmp_v[pl.ds(w * 16, 16)]
        scal_v[pl.ds(0, 16)] = (jnp.zeros((16,), jnp.float32) + jnp.sum(t)) * 0.0625
        pltpu.sync_copy(scal_v.at[pl.ds(0, 16)], out_hbm)


@jax.jit
def kernel(x, y, x_weights, y_weights):
    mesh = plsc.VectorSubcoreMesh(core_axis_name="c", subcore_axis_name="s",
                                  num_cores=1)
    run = pl.kernel(
        _body,
        out_type=jax.ShapeDtypeStruct((16,), jnp.float32),
        mesh=mesh,
        compiler_params=pltpu.CompilerParams(needs_layout_passes=False),
        scratch_types=[
            pltpu.VMEM((CHUNK + 16,), jnp.int32),
            pltpu.VMEM((CHUNK,), jnp.float32),
            pltpu.VMEM((CHUNK // 128, 128), jnp.int32),
            pltpu.VMEM((CHUNK,), jnp.int32),
            pltpu.VMEM((CHUNK,), jnp.float32),
            pltpu.VMEM((RADIX * 16,), jnp.int32),
            pltpu.VMEM((RADIX,), jnp.int32),
            pltpu.VMEM((512,), jnp.float32),
            pltpu.VMEM((NW * RADIX,), jnp.int32),
            pltpu.VMEM((16,), jnp.float32),
            pltpu.VMEM_SHARED((N2 + PAD,), jnp.int32),
            pltpu.VMEM_SHARED((N2,), jnp.float32),
            pltpu.VMEM_SHARED((NW * RADIX,), jnp.int32),
            pltpu.VMEM_SHARED((NW * 16,), jnp.float32),
            pltpu.SemaphoreType.DMA,
        ],
    )
    out = run(x, y, x_weights, y_weights)
    return out[0].reshape(())
